# baseline (speedup 1.0000x reference)
import functools

import jax
import jax.numpy as jnp
from jax import lax
from jax.experimental import pallas as pl
from jax.experimental.pallas import tpu as pltpu
from jax.experimental.pallas import tpu_sc as plsc

N_NODES = 10000
N_EDGES = 320000
D = 128

NC = 2
NS = 16
CH = 128
KPT = 160
TOTCH = NS * KPT
E_PAD = TOTCH * CH
NP = 10240
RPT = NP // NS
BM = 512
GRID_M = NP // BM

_mesh = plsc.VectorSubcoreMesh(core_axis_name="c", subcore_axis_name="s")

_SC_SCRATCH = [
    pltpu.VMEM((2, CH), jnp.int32),
    pltpu.VMEM((2, CH), jnp.int32),
    pltpu.VMEM((CH, D), jnp.float32),
    pltpu.VMEM((CH, D), jnp.float32),
    pltpu.VMEM_SHARED((NP, D), jnp.float32),
    pltpu.SemaphoreType.DMA,
    pltpu.SemaphoreType.DMA,
    pltpu.SemaphoreType.DMA,
    pltpu.SemaphoreType.DMA,
]


def _agg_loop(y_hbm, idx_hbm, off, idxA, idxB, rowsA, rowsB, agg_sh,
              gsA, gsB, ssA, ssB):
    def pair(p, carry):
        @pl.when(p > 0)
        def _():
            pltpu.make_async_copy(rowsA, agg_sh.at[idxA.at[1]], ssA).wait()
            pltpu.make_async_copy(rowsB, agg_sh.at[idxB.at[1]], ssB).wait()
        t = off + 2 * p
        pltpu.sync_copy(idx_hbm.at[t], idxA)
        gA = pltpu.async_copy(y_hbm.at[idxA.at[0]], rowsA, gsA)
        pltpu.sync_copy(idx_hbm.at[t + 1], idxB)
        gB = pltpu.async_copy(y_hbm.at[idxB.at[0]], rowsB, gsB)
        gA.wait()
        pltpu.async_copy(rowsA, agg_sh.at[idxA.at[1]], ssA, add=True)
        gB.wait()
        pltpu.async_copy(rowsB, agg_sh.at[idxB.at[1]], ssB, add=True)
        return carry

    lax.fori_loop(0, KPT // 2, pair, 0)
    pltpu.make_async_copy(rowsA, agg_sh.at[idxA.at[1]], ssA).wait()
    pltpu.make_async_copy(rowsB, agg_sh.at[idxB.at[1]], ssB).wait()


@functools.partial(
    pl.kernel,
    out_type=[jax.ShapeDtypeStruct((NP, D), jnp.float32),
              jax.ShapeDtypeStruct((NP, D), jnp.float32)],
    mesh=_mesh,
    scratch_types=_SC_SCRATCH,
)
def _sc_l1(y_hbm, idx_hbm, zeros_hbm, ones_hbm, agg_out, cnt_out,
           idxA, idxB, rowsA, rowsB, agg_sh, gsA, gsB, ssA, ssB):
    c = lax.axis_index("c")
    s = lax.axis_index("s")
    rp = s * RPT
    off = s * KPT
    pltpu.sync_copy(zeros_hbm.at[pl.ds(rp, RPT)], agg_sh.at[pl.ds(rp, RPT)])

    @pl.when(c == 1)
    def _():
        pltpu.sync_copy(ones_hbm, rowsA)

    plsc.subcore_barrier()

    @pl.when(c == 0)
    def _():
        _agg_loop(y_hbm, idx_hbm, off, idxA, idxB, rowsA, rowsB, agg_sh,
                  gsA, gsB, ssA, ssB)

    @pl.when(c == 1)
    def _():
        def body(j, carry):
            pltpu.sync_copy(idx_hbm.at[off + j], idxA)
            pltpu.sync_copy(rowsA, agg_sh.at[idxA.at[1]], add=True)
            return carry

        lax.fori_loop(0, KPT, body, 0)

    plsc.subcore_barrier()

    @pl.when(c == 0)
    def _():
        pltpu.sync_copy(agg_sh.at[pl.ds(rp, RPT)], agg_out.at[pl.ds(rp, RPT)])

    @pl.when(c == 1)
    def _():
        pltpu.sync_copy(agg_sh.at[pl.ds(rp, RPT)], cnt_out.at[pl.ds(rp, RPT)])


@functools.partial(
    pl.kernel,
    out_type=[jax.ShapeDtypeStruct((NP, D), jnp.float32)],
    mesh=_mesh,
    scratch_types=_SC_SCRATCH,
)
def _sc_l2(y_hbm, idx_hbm, zeros_hbm, agg_out,
           idxA, idxB, rowsA, rowsB, agg_sh, gsA, gsB, ssA, ssB):
    c = lax.axis_index("c")
    s = lax.axis_index("s")
    rp = s * RPT
    off = s * KPT

    @pl.when(c == 0)
    def _():
        pltpu.sync_copy(zeros_hbm.at[pl.ds(rp, RPT)],
                        agg_sh.at[pl.ds(rp, RPT)])

    plsc.subcore_barrier()

    @pl.when(c == 0)
    def _():
        _agg_loop(y_hbm, idx_hbm, off, idxA, idxB, rowsA, rowsB, agg_sh,
                  gsA, gsB, ssA, ssB)

    plsc.subcore_barrier()

    @pl.when(c == 0)
    def _():
        pltpu.sync_copy(agg_sh.at[pl.ds(rp, RPT)], agg_out.at[pl.ds(rp, RPT)])


def _tc1_body(x_ref, wl_ref, wr_ref, y_ref, z_ref):
    xb = x_ref[...]
    y_ref[...] = jnp.dot(xb, wl_ref[...], preferred_element_type=jnp.float32)
    z_ref[...] = jnp.dot(xb, wr_ref[...], preferred_element_type=jnp.float32)


def _tc2_body(a_ref, c_ref, z_ref, b_ref, wl_ref, wr_ref, y2_ref, z2_ref):
    denom = jnp.maximum(c_ref[...], 1.0)
    h = a_ref[...] / denom + b_ref[...] + z_ref[...]
    h = jnp.maximum(h, 0.0)
    y2_ref[...] = jnp.dot(h, wl_ref[...], preferred_element_type=jnp.float32)
    z2_ref[...] = jnp.dot(h, wr_ref[...], preferred_element_type=jnp.float32)


def _tc3_body(a_ref, c_ref, z_ref, b_ref, out_ref):
    denom = jnp.maximum(c_ref[...], 1.0)
    out_ref[...] = a_ref[...] / denom + b_ref[...] + z_ref[...]


_row_spec = pl.BlockSpec((BM, D), lambda i: (i, 0))
_w_spec = pl.BlockSpec((D, D), lambda i: (0, 0))
_b_spec = pl.BlockSpec((1, D), lambda i: (0, 0))

_tc1 = pl.pallas_call(
    _tc1_body,
    grid=(GRID_M,),
    in_specs=[_row_spec, _w_spec, _w_spec],
    out_specs=[_row_spec, _row_spec],
    out_shape=[jax.ShapeDtypeStruct((NP, D), jnp.float32)] * 2,
)

_tc2 = pl.pallas_call(
    _tc2_body,
    grid=(GRID_M,),
    in_specs=[_row_spec, _row_spec, _row_spec, _b_spec, _w_spec, _w_spec],
    out_specs=[_row_spec, _row_spec],
    out_shape=[jax.ShapeDtypeStruct((NP, D), jnp.float32)] * 2,
)

_tc3 = pl.pallas_call(
    _tc3_body,
    grid=(GRID_M,),
    in_specs=[_row_spec, _row_spec, _row_spec, _b_spec],
    out_specs=_row_spec,
    out_shape=jax.ShapeDtypeStruct((NP, D), jnp.float32),
)


@jax.jit
def kernel(x, edge_index, W1_l, W1_r, b1, W2_l, W2_r, b2):
    src = edge_index[0].astype(jnp.int32)
    dst = edge_index[1].astype(jnp.int32)
    pad = E_PAD - N_EDGES
    srcp = jnp.concatenate([src, jnp.zeros((pad,), jnp.int32)]).reshape(
        TOTCH, CH)
    dstp = jnp.concatenate([dst, jnp.full((pad,), N_NODES, jnp.int32)]
                           ).reshape(TOTCH, CH)
    idx2 = jnp.stack([srcp, dstp], axis=1)
    xp = jnp.pad(x, ((0, NP - N_NODES), (0, 0)))
    zeros = jnp.zeros((NP, D), jnp.float32)
    ones = jnp.ones((CH, D), jnp.float32)
    b1r = b1.reshape(1, D)
    b2r = b2.reshape(1, D)

    y1, z1 = _tc1(xp, W1_l, W1_r)
    agg1, cnt = _sc_l1(y1, idx2, zeros, ones)
    y2, z2 = _tc2(agg1, cnt, z1, b1r, W2_l, W2_r)
    (agg2,) = _sc_l2(y2, idx2, zeros)
    out = _tc3(agg2, cnt, z2, b2r)
    return out[:N_NODES]


# speedup vs baseline: 1.1986x; 1.1986x over previous
import functools

import jax
import jax.numpy as jnp
from jax import lax
from jax.experimental import pallas as pl
from jax.experimental.pallas import tpu as pltpu
from jax.experimental.pallas import tpu_sc as plsc

N_NODES = 10000
N_EDGES = 320000
D = 128

NC = 2
NS = 16
NW = NC * NS
CH = 128
KPW = 80
TOTCH = NW * KPW
K0 = 132
K1 = 28
E_PAD = TOTCH * CH
NP = 10240
RPT = NP // NS
BM = 512
GRID_M = NP // BM

_mesh = plsc.VectorSubcoreMesh(core_axis_name="c", subcore_axis_name="s")


@functools.partial(
    pl.kernel,
    out_type=[jax.ShapeDtypeStruct((NC, NP, D), jnp.float32)],
    mesh=_mesh,
    scratch_types=[
        pltpu.VMEM((2, CH), jnp.int32),
        pltpu.VMEM((2, CH), jnp.int32),
        pltpu.VMEM((CH, D), jnp.float32),
        pltpu.VMEM((CH, D), jnp.float32),
        pltpu.VMEM_SHARED((NP, D), jnp.float32),
        pltpu.SemaphoreType.DMA,
        pltpu.SemaphoreType.DMA,
        pltpu.SemaphoreType.DMA,
        pltpu.SemaphoreType.DMA,
    ],
)
def _sc_agg(y_hbm, idx_hbm, zeros_hbm, agg_out,
            idxA, idxB, rowsA, rowsB, agg_sh, gsA, gsB, ssA, ssB):
    c = lax.axis_index("c")
    s = lax.axis_index("s")
    rp = s * RPT
    off = lax.select(c == 0, s * K0, 16 * K0 + s * K1)
    npairs = lax.select(c == 0, K0 // 2, K1 // 2)
    pltpu.sync_copy(zeros_hbm.at[pl.ds(rp, RPT)], agg_sh.at[pl.ds(rp, RPT)])
    plsc.subcore_barrier()

    def pair(p, carry):
        @pl.when(p > 0)
        def _():
            pltpu.make_async_copy(rowsA, agg_sh.at[idxA.at[1]], ssA).wait()
            pltpu.make_async_copy(rowsB, agg_sh.at[idxB.at[1]], ssB).wait()
        t = off + 2 * p
        pltpu.sync_copy(idx_hbm.at[t], idxA)
        gA = pltpu.async_copy(y_hbm.at[idxA.at[0]], rowsA, gsA)
        pltpu.sync_copy(idx_hbm.at[t + 1], idxB)
        gB = pltpu.async_copy(y_hbm.at[idxB.at[0]], rowsB, gsB)
        gA.wait()
        pltpu.async_copy(rowsA, agg_sh.at[idxA.at[1]], ssA, add=True)
        gB.wait()
        pltpu.async_copy(rowsB, agg_sh.at[idxB.at[1]], ssB, add=True)
        return carry

    lax.fori_loop(0, npairs, pair, 0)
    pltpu.make_async_copy(rowsA, agg_sh.at[idxA.at[1]], ssA).wait()
    pltpu.make_async_copy(rowsB, agg_sh.at[idxB.at[1]], ssB).wait()
    plsc.subcore_barrier()
    pltpu.sync_copy(agg_sh.at[pl.ds(rp, RPT)], agg_out.at[c, pl.ds(rp, RPT)])


@functools.partial(
    pl.kernel,
    out_type=[jax.ShapeDtypeStruct((NC, NP, D), jnp.float32)],
    mesh=_mesh,
    scratch_types=[
        pltpu.VMEM((KPW, CH), jnp.int32),
        pltpu.VMEM((CH, D), jnp.float32),
        pltpu.VMEM_SHARED((NP, D), jnp.float32),
    ],
)
def _sc_cnt(dst_hbm, zeros_hbm, ones_hbm, cnt_out, dst_v, ones_v, cnt_sh):
    c = lax.axis_index("c")
    s = lax.axis_index("s")
    wid = c * NS + s
    rp = s * RPT
    pltpu.sync_copy(zeros_hbm.at[pl.ds(rp, RPT)], cnt_sh.at[pl.ds(rp, RPT)])
    pltpu.sync_copy(dst_hbm.at[pl.ds(wid * KPW, KPW)], dst_v)
    pltpu.sync_copy(ones_hbm, ones_v)
    plsc.subcore_barrier()

    def body(j, carry):
        pltpu.sync_copy(ones_v, cnt_sh.at[dst_v.at[j]], add=True)
        return carry

    lax.fori_loop(0, KPW, body, 0)
    plsc.subcore_barrier()
    pltpu.sync_copy(cnt_sh.at[pl.ds(rp, RPT)], cnt_out.at[c, pl.ds(rp, RPT)])


def _tc1_body(x_ref, wl_ref, wr_ref, y_ref, z_ref):
    xb = x_ref[...]
    y_ref[...] = jnp.dot(xb, wl_ref[...], preferred_element_type=jnp.float32)
    z_ref[...] = jnp.dot(xb, wr_ref[...], preferred_element_type=jnp.float32)


def _tc2_body(a_ref, c_ref, z_ref, b_ref, wl_ref, wr_ref, y2_ref, z2_ref):
    denom = jnp.maximum(c_ref[0] + c_ref[1], 1.0)
    h = (a_ref[0] + a_ref[1]) / denom + b_ref[...] + z_ref[...]
    h = jnp.maximum(h, 0.0)
    y2_ref[...] = jnp.dot(h, wl_ref[...], preferred_element_type=jnp.float32)
    z2_ref[...] = jnp.dot(h, wr_ref[...], preferred_element_type=jnp.float32)


def _tc3_body(a_ref, c_ref, z_ref, b_ref, out_ref):
    denom = jnp.maximum(c_ref[0] + c_ref[1], 1.0)
    out_ref[...] = (a_ref[0] + a_ref[1]) / denom + b_ref[...] + z_ref[...]


_row_spec = pl.BlockSpec((BM, D), lambda i: (i, 0))
_par_spec = pl.BlockSpec((NC, BM, D), lambda i: (0, i, 0))
_w_spec = pl.BlockSpec((D, D), lambda i: (0, 0))
_b_spec = pl.BlockSpec((1, D), lambda i: (0, 0))

_tc1 = pl.pallas_call(
    _tc1_body,
    grid=(GRID_M,),
    in_specs=[_row_spec, _w_spec, _w_spec],
    out_specs=[_row_spec, _row_spec],
    out_shape=[jax.ShapeDtypeStruct((NP, D), jnp.float32)] * 2,
)

_tc2 = pl.pallas_call(
    _tc2_body,
    grid=(GRID_M,),
    in_specs=[_par_spec, _par_spec, _row_spec, _b_spec, _w_spec, _w_spec],
    out_specs=[_row_spec, _row_spec],
    out_shape=[jax.ShapeDtypeStruct((NP, D), jnp.float32)] * 2,
)

_tc3 = pl.pallas_call(
    _tc3_body,
    grid=(GRID_M,),
    in_specs=[_par_spec, _par_spec, _row_spec, _b_spec],
    out_specs=_row_spec,
    out_shape=jax.ShapeDtypeStruct((NP, D), jnp.float32),
)


@jax.jit
def kernel(x, edge_index, W1_l, W1_r, b1, W2_l, W2_r, b2):
    src = edge_index[0].astype(jnp.int32)
    dst = edge_index[1].astype(jnp.int32)
    pad = E_PAD - N_EDGES
    srcp = jnp.concatenate([src, jnp.zeros((pad,), jnp.int32)]).reshape(
        TOTCH, CH)
    dstp = jnp.concatenate([dst, jnp.full((pad,), N_NODES, jnp.int32)]
                           ).reshape(TOTCH, CH)
    idx2 = jnp.stack([srcp, dstp], axis=1)
    xp = jnp.pad(x, ((0, NP - N_NODES), (0, 0)))
    zeros = jnp.zeros((NP, D), jnp.float32)
    ones = jnp.ones((CH, D), jnp.float32)
    b1r = b1.reshape(1, D)
    b2r = b2.reshape(1, D)

    (cntp,) = _sc_cnt(dstp, zeros, ones)
    y1, z1 = _tc1(xp, W1_l, W1_r)
    (aggp1,) = _sc_agg(y1, idx2, zeros)
    y2, z2 = _tc2(aggp1, cntp, z1, b1r, W2_l, W2_r)
    (aggp2,) = _sc_agg(y2, idx2, zeros)
    out = _tc3(aggp2, cntp, z2, b2r)
    return out[:N_NODES]


# speedup vs baseline: 1.4336x; 1.1961x over previous
import functools

import jax
import jax.numpy as jnp
from jax import lax
from jax.experimental import pallas as pl
from jax.experimental.pallas import tpu as pltpu
from jax.experimental.pallas import tpu_sc as plsc

N_NODES = 10000
N_EDGES = 320000
D = 128

NC = 2
NS = 16
NW = NC * NS
CH = 128
KPW = 80
TOTCH = NW * KPW
K0 = 132
K1 = 28
E_PAD = TOTCH * CH
NP = 10240
RPT = NP // NS
BM = 512
GRID_M = NP // BM

_mesh = plsc.VectorSubcoreMesh(core_axis_name="c", subcore_axis_name="s")


@functools.partial(
    pl.kernel,
    out_type=[jax.ShapeDtypeStruct((NC, NP, D), jnp.float32)],
    mesh=_mesh,
    scratch_types=[
        pltpu.VMEM((2, CH), jnp.int32),
        pltpu.VMEM((2, CH), jnp.int32),
        pltpu.VMEM((CH, D), jnp.float32),
        pltpu.VMEM((CH, D), jnp.float32),
        pltpu.VMEM_SHARED((NP, D), jnp.float32),
        pltpu.SemaphoreType.DMA,
        pltpu.SemaphoreType.DMA,
        pltpu.SemaphoreType.DMA,
        pltpu.SemaphoreType.DMA,
    ],
)
def _sc_agg(y_hbm, idx_hbm, zeros_hbm, agg_out,
            idxA, idxB, rowsA, rowsB, agg_sh, gsA, gsB, ssA, ssB):
    c = lax.axis_index("c")
    s = lax.axis_index("s")
    rp = s * RPT
    off = lax.select(c == 0, s * K0, 16 * K0 + s * K1)
    npairs = lax.select(c == 0, K0 // 2, K1 // 2)
    pltpu.sync_copy(zeros_hbm.at[pl.ds(rp, RPT)], agg_sh.at[pl.ds(rp, RPT)])
    plsc.subcore_barrier()

    def pair(p, carry):
        @pl.when(p > 0)
        def _():
            pltpu.make_async_copy(rowsA, agg_sh.at[idxA.at[1]], ssA).wait()
            pltpu.make_async_copy(rowsB, agg_sh.at[idxB.at[1]], ssB).wait()
        t = off + 2 * p
        pltpu.sync_copy(idx_hbm.at[t], idxA)
        gA = pltpu.async_copy(y_hbm.at[idxA.at[0]], rowsA, gsA)
        pltpu.sync_copy(idx_hbm.at[t + 1], idxB)
        gB = pltpu.async_copy(y_hbm.at[idxB.at[0]], rowsB, gsB)
        gA.wait()
        pltpu.async_copy(rowsA, agg_sh.at[idxA.at[1]], ssA, add=True)
        gB.wait()
        pltpu.async_copy(rowsB, agg_sh.at[idxB.at[1]], ssB, add=True)
        return carry

    lax.fori_loop(0, npairs, pair, 0)
    pltpu.make_async_copy(rowsA, agg_sh.at[idxA.at[1]], ssA).wait()
    pltpu.make_async_copy(rowsB, agg_sh.at[idxB.at[1]], ssB).wait()
    plsc.subcore_barrier()
    pltpu.sync_copy(agg_sh.at[pl.ds(rp, RPT)], agg_out.at[c, pl.ds(rp, RPT)])


@functools.partial(
    pl.kernel,
    out_type=[jax.ShapeDtypeStruct((NC, NP, D), jnp.float32)],
    mesh=_mesh,
    scratch_types=[
        pltpu.VMEM((2, CH), jnp.int32),
        pltpu.VMEM((2, CH), jnp.int32),
        pltpu.VMEM((CH, D), jnp.float32),
        pltpu.VMEM_SHARED((NP, D), jnp.float32),
        pltpu.SemaphoreType.DMA,
        pltpu.SemaphoreType.DMA,
    ],
)
def _sc_cnt(idx_hbm, zeros_hbm, ones_hbm, cnt_out,
            idxA, idxB, ones_v, cnt_sh, ssA, ssB):
    c = lax.axis_index("c")
    s = lax.axis_index("s")
    wid = c * NS + s
    rp = s * RPT
    off = wid * KPW
    pltpu.sync_copy(zeros_hbm.at[pl.ds(rp, RPT)], cnt_sh.at[pl.ds(rp, RPT)])
    pltpu.sync_copy(ones_hbm, ones_v)
    plsc.subcore_barrier()

    def pair(p, carry):
        @pl.when(p > 0)
        def _():
            pltpu.make_async_copy(ones_v, cnt_sh.at[idxA.at[1]], ssA).wait()
            pltpu.make_async_copy(ones_v, cnt_sh.at[idxB.at[1]], ssB).wait()
        t = off + 2 * p
        pltpu.sync_copy(idx_hbm.at[t], idxA)
        pltpu.async_copy(ones_v, cnt_sh.at[idxA.at[1]], ssA, add=True)
        pltpu.sync_copy(idx_hbm.at[t + 1], idxB)
        pltpu.async_copy(ones_v, cnt_sh.at[idxB.at[1]], ssB, add=True)
        return carry

    lax.fori_loop(0, KPW // 2, pair, 0)
    pltpu.make_async_copy(ones_v, cnt_sh.at[idxA.at[1]], ssA).wait()
    pltpu.make_async_copy(ones_v, cnt_sh.at[idxB.at[1]], ssB).wait()
    plsc.subcore_barrier()
    pltpu.sync_copy(cnt_sh.at[pl.ds(rp, RPT)], cnt_out.at[c, pl.ds(rp, RPT)])


def _tc1_body(x_ref, wl_ref, wr_ref, y_ref, z_ref):
    xb = x_ref[...]
    y_ref[...] = jnp.dot(xb, wl_ref[...], preferred_element_type=jnp.float32)
    z_ref[...] = jnp.dot(xb, wr_ref[...], preferred_element_type=jnp.float32)


def _tc2_body(a_ref, c_ref, z_ref, b_ref, wl_ref, wr_ref, y2_ref, z2_ref):
    denom = jnp.maximum(c_ref[0] + c_ref[1], 1.0)
    h = (a_ref[0] + a_ref[1]) / denom + b_ref[...] + z_ref[...]
    h = jnp.maximum(h, 0.0)
    y2_ref[...] = jnp.dot(h, wl_ref[...], preferred_element_type=jnp.float32)
    z2_ref[...] = jnp.dot(h, wr_ref[...], preferred_element_type=jnp.float32)


def _tc3_body(a_ref, c_ref, z_ref, b_ref, out_ref):
    denom = jnp.maximum(c_ref[0] + c_ref[1], 1.0)
    out_ref[...] = (a_ref[0] + a_ref[1]) / denom + b_ref[...] + z_ref[...]


_row_spec = pl.BlockSpec((BM, D), lambda i: (i, 0))
_par_spec = pl.BlockSpec((NC, BM, D), lambda i: (0, i, 0))
_w_spec = pl.BlockSpec((D, D), lambda i: (0, 0))
_b_spec = pl.BlockSpec((1, D), lambda i: (0, 0))

_tc1 = pl.pallas_call(
    _tc1_body,
    grid=(GRID_M,),
    in_specs=[_row_spec, _w_spec, _w_spec],
    out_specs=[_row_spec, _row_spec],
    out_shape=[jax.ShapeDtypeStruct((NP, D), jnp.float32)] * 2,
)

_tc2 = pl.pallas_call(
    _tc2_body,
    grid=(GRID_M,),
    in_specs=[_par_spec, _par_spec, _row_spec, _b_spec, _w_spec, _w_spec],
    out_specs=[_row_spec, _row_spec],
    out_shape=[jax.ShapeDtypeStruct((NP, D), jnp.float32)] * 2,
)

_tc3 = pl.pallas_call(
    _tc3_body,
    grid=(GRID_M,),
    in_specs=[_par_spec, _par_spec, _row_spec, _b_spec],
    out_specs=_row_spec,
    out_shape=jax.ShapeDtypeStruct((NP, D), jnp.float32),
)


@jax.jit
def kernel(x, edge_index, W1_l, W1_r, b1, W2_l, W2_r, b2):
    src = edge_index[0].astype(jnp.int32)
    dst = edge_index[1].astype(jnp.int32)
    pad = E_PAD - N_EDGES
    srcp = jnp.concatenate([src, jnp.zeros((pad,), jnp.int32)]).reshape(
        TOTCH, CH)
    dstp = jnp.concatenate([dst, jnp.full((pad,), N_NODES, jnp.int32)]
                           ).reshape(TOTCH, CH)
    idx2 = jnp.stack([srcp, dstp], axis=1)
    xp = jnp.pad(x, ((0, NP - N_NODES), (0, 0)))
    zeros = jnp.zeros((NP, D), jnp.float32)
    ones = jnp.ones((CH, D), jnp.float32)
    b1r = b1.reshape(1, D)
    b2r = b2.reshape(1, D)

    (cntp,) = _sc_cnt(idx2, zeros, ones)
    y1, z1 = _tc1(xp, W1_l, W1_r)
    (aggp1,) = _sc_agg(y1, idx2, zeros)
    y2, z2 = _tc2(aggp1, cntp, z1, b1r, W2_l, W2_r)
    (aggp2,) = _sc_agg(y2, idx2, zeros)
    out = _tc3(aggp2, cntp, z2, b2r)
    return out[:N_NODES]
